# baseline (speedup 1.0000x reference)
import functools

import jax
import jax.numpy as jnp
from jax import lax
from jax.experimental import pallas as pl
from jax.experimental.pallas import tpu as pltpu
from jax.experimental.pallas import tpu_sc as plsc

N = 10000
E = 320000
B = 100
NPG = 100
DX = 96
MAXDEG = 16
RW = 16
F = 128
TAU = 0.005
SK_ITERS = 10

NC = 2
NS = 16
NW = NC * NS
EPW = E // NW
CH = 100
NCH = EPW // CH
NT = NCH // 2
NTT = NCH // 3
RPS = 624
TAIL = N - NS * RPS

def _mesh():
    return plsc.VectorSubcoreMesh(core_axis_name="c", subcore_axis_name="s",
                                  num_cores=NC, num_subcores=NS)


def _deg_body(dsts_hbm, z_hbm, out_hbm, idx_v, ones_v, acc1, acc2,
              sem_a, sem_b):
    cid = lax.axis_index("c")
    sid = lax.axis_index("s")
    wid = sid * NC + cid

    @pl.when(sid == 0)
    def _zero():
        pltpu.sync_copy(z_hbm, acc1)
        pltpu.sync_copy(z_hbm, acc2)

    def _fill_ones(i, carry):
        ones_v[i] = jnp.ones((16,), jnp.float32)
        return carry

    lax.fori_loop(0, CH, _fill_ones, 0)
    plsc.subcore_barrier()

    for g, acc in ((0, acc1), (1, acc2)):
        pltpu.sync_copy(dsts_hbm.at[g, wid], idx_v)

        def _step(t, carry, acc=acc):
            @pl.when(t > 0)
            def _wa():
                pltpu.make_async_copy(
                    ones_v, acc.at[idx_v.at[2 * t - 2]], sem_a).wait()

            pltpu.async_copy(ones_v, acc.at[idx_v.at[2 * t]], sem_a,
                             add=True)

            @pl.when(t > 0)
            def _wb():
                pltpu.make_async_copy(
                    ones_v, acc.at[idx_v.at[2 * t - 1]], sem_b).wait()

            pltpu.async_copy(ones_v, acc.at[idx_v.at[2 * t + 1]], sem_b,
                             add=True)
            return carry

        lax.fori_loop(0, NT, _step, 0)
        pltpu.make_async_copy(ones_v, acc.at[idx_v.at[NCH - 2]],
                              sem_a).wait()
        pltpu.make_async_copy(ones_v, acc.at[idx_v.at[NCH - 1]],
                              sem_b).wait()

    plsc.subcore_barrier()
    base = pl.multiple_of(sid * RPS, 8)
    pltpu.sync_copy(acc1.at[pl.ds(base, RPS)],
                    out_hbm.at[cid, 0, pl.ds(base, RPS)])
    pltpu.sync_copy(acc2.at[pl.ds(base, RPS)],
                    out_hbm.at[cid, 1, pl.ds(base, RPS)])

    @pl.when(sid == 0)
    def _tail():
        pltpu.sync_copy(acc1.at[pl.ds(NS * RPS, TAIL)],
                        out_hbm.at[cid, 0, pl.ds(NS * RPS, TAIL)])
        pltpu.sync_copy(acc2.at[pl.ds(NS * RPS, TAIL)],
                        out_hbm.at[cid, 1, pl.ds(NS * RPS, TAIL)])


def _sc_degrees(dsts, z16):
    return pl.kernel(
        _deg_body,
        out_type=jax.ShapeDtypeStruct((NC, 2, N, 16), jnp.float32),
        mesh=_mesh(),
        scratch_types=[
            pltpu.VMEM((NCH, CH), jnp.int32),
            pltpu.VMEM((CH, 16), jnp.float32),
            pltpu.VMEM_SHARED((N, 16), jnp.float32),
            pltpu.VMEM_SHARED((N, 16), jnp.float32),
            pltpu.SemaphoreType.DMA,
            pltpu.SemaphoreType.DMA,
        ],
    )(dsts, z16)


def _scatter_body(y_hbm, sd_hbm, z_hbm, out_hbm,
                  idx0, idx1, idx2, rows0, rows1, rows2, acc,
                  gs0, gs1, gs2, ss0, ss1, ss2, is0, is1, is2):
    cid = lax.axis_index("c")
    sid = lax.axis_index("s")
    wid = sid * NC + cid
    idxb = (idx0, idx1, idx2)
    rows = (rows0, rows1, rows2)
    gs = (gs0, gs1, gs2)
    ss = (ss0, ss1, ss2)
    iss = (is0, is1, is2)

    @pl.when(sid == 0)
    def _zero():
        pltpu.sync_copy(z_hbm, acc)

    pltpu.async_copy(sd_hbm.at[wid, pl.ds(0, 1)], idx0, is0)
    pltpu.async_copy(sd_hbm.at[wid, pl.ds(1, 1)], idx1, is1)
    plsc.subcore_barrier()
    pltpu.make_async_copy(sd_hbm.at[wid, pl.ds(0, 1)], idx0, is0).wait()
    pltpu.async_copy(y_hbm.at[idx0.at[0, 0]], rows0, gs0)
    pltpu.make_async_copy(sd_hbm.at[wid, pl.ds(1, 1)], idx1, is1).wait()
    pltpu.async_copy(y_hbm.at[idx1.at[0, 0]], rows1, gs1)

    def _chunk(j, t, b, guard_first, guard_last):
        bn = (b + 2) % 3
        pltpu.make_async_copy(y_hbm.at[idxb[b].at[0, 0]], rows[b],
                              gs[b]).wait()
        pltpu.async_copy(rows[b], acc.at[idxb[b].at[0, 1]], ss[b], add=True)

        def _free_and_prefetch():
            pltpu.make_async_copy(rows[bn], acc.at[idxb[bn].at[0, 1]],
                                  ss[bn]).wait()
            pltpu.async_copy(sd_hbm.at[wid, pl.ds(j + 2, 1)], idxb[bn],
                             iss[bn])
            pltpu.make_async_copy(sd_hbm.at[wid, pl.ds(j + 2, 1)],
                                  idxb[bn], iss[bn]).wait()
            pltpu.async_copy(y_hbm.at[idxb[bn].at[0, 0]], rows[bn], gs[bn])

        if guard_first:
            @pl.when(t > 0)
            def _a():
                _free_and_prefetch()

            @pl.when(t == 0)
            def _b():
                pltpu.async_copy(sd_hbm.at[wid, pl.ds(j + 2, 1)],
                                 idxb[bn], iss[bn])
                pltpu.make_async_copy(sd_hbm.at[wid, pl.ds(j + 2, 1)],
                                      idxb[bn], iss[bn]).wait()
                pltpu.async_copy(y_hbm.at[idxb[bn].at[0, 0]], rows[bn],
                                 gs[bn])
        elif guard_last:
            @pl.when(t < NTT - 1)
            def _c():
                _free_and_prefetch()

            @pl.when(t == NTT - 1)
            def _d():
                pltpu.make_async_copy(rows[bn], acc.at[idxb[bn].at[0, 1]],
                                      ss[bn]).wait()
        else:
            _free_and_prefetch()

    def _step(t, carry):
        j0 = 3 * t
        _chunk(j0, t, 0, True, False)
        _chunk(j0 + 1, t, 1, False, False)
        _chunk(j0 + 2, t, 2, False, True)
        return carry

    lax.fori_loop(0, NTT, _step, 0)
    pltpu.make_async_copy(y_hbm.at[idx0.at[0, 0]], rows0, gs0).wait()
    pltpu.async_copy(rows0, acc.at[idx0.at[0, 1]], ss0, add=True)
    pltpu.make_async_copy(rows2, acc.at[idx2.at[0, 1]], ss2).wait()
    pltpu.make_async_copy(rows0, acc.at[idx0.at[0, 1]], ss0).wait()

    plsc.subcore_barrier()
    base = pl.multiple_of(sid * RPS, 8)
    pltpu.sync_copy(acc.at[pl.ds(base, RPS)],
                    out_hbm.at[cid, pl.ds(base, RPS)])

    @pl.when(sid == 0)
    def _tail():
        pltpu.sync_copy(acc.at[pl.ds(NS * RPS, TAIL)],
                        out_hbm.at[cid, pl.ds(NS * RPS, TAIL)])


def _sc_scatter_rows(y, sd, z128):
    return pl.kernel(
        _scatter_body,
        out_type=jax.ShapeDtypeStruct((NC, N, F), jnp.float32),
        mesh=_mesh(),
        scratch_types=(
            [pltpu.VMEM((1, 2, CH), jnp.int32)] * 3
            + [pltpu.VMEM((CH, F), jnp.float32)] * 3
            + [pltpu.VMEM_SHARED((N, F), jnp.float32)]
            + [pltpu.SemaphoreType.DMA] * 9
        ),
    )(y, sd, z128)


BT = 2000
_PREC = lax.Precision.HIGHEST


def _dinv_of(deg2_blk):
    d = deg2_blk[0, :, 0] + deg2_blk[1, :, 0] + 1.0
    return lax.rsqrt(d)


def _init_body(x_ref, cent_ref, rw_ref, deg2_ref, demb_ref, wi_ref, bi_ref,
               wg_ref, xw_ref, y_ref):
    dinv = _dinv_of(deg2_ref[...])
    iot = lax.broadcasted_iota(jnp.int32, (1, MAXDEG), 1)
    oh = (cent_ref[...] == iot).astype(jnp.float32)
    h = jnp.dot(x_ref[...], wi_ref[0:DX, :], precision=_PREC)
    h += jnp.dot(jnp.dot(oh, demb_ref[...], precision=_PREC),
                 wi_ref[DX:DX + MAXDEG, :], precision=_PREC)
    h += jnp.dot(rw_ref[...], wi_ref[DX + MAXDEG:, :], precision=_PREC)
    h = jax.nn.relu(h + bi_ref[...])
    xw = jnp.dot(h, wg_ref[...], precision=_PREC)
    xw_ref[...] = xw
    y_ref[...] = xw * dinv[:, None]


def _tc_init(x, cent, rw, deg2, demb, w_init, b_init, wg1):
    grid = (N // BT,)
    return pl.pallas_call(
        _init_body,
        grid=grid,
        in_specs=[
            pl.BlockSpec((BT, DX), lambda i: (i, 0)),
            pl.BlockSpec((BT, 1), lambda i: (i, 0)),
            pl.BlockSpec((BT, RW), lambda i: (i, 0)),
            pl.BlockSpec((2, BT, 16), lambda i: (0, i, 0)),
            pl.BlockSpec((MAXDEG, MAXDEG), lambda i: (0, 0)),
            pl.BlockSpec((DX + MAXDEG + RW, F), lambda i: (0, 0)),
            pl.BlockSpec((1, F), lambda i: (0, 0)),
            pl.BlockSpec((F, F), lambda i: (0, 0)),
        ],
        out_specs=[
            pl.BlockSpec((BT, F), lambda i: (i, 0)),
            pl.BlockSpec((BT, F), lambda i: (i, 0)),
        ],
        out_shape=[
            jax.ShapeDtypeStruct((N, F), jnp.float32),
            jax.ShapeDtypeStruct((N, F), jnp.float32),
        ],
    )(x, cent, rw, deg2, demb, w_init, b_init, wg1)


def _step_body(s_ref, xw_ref, deg2_ref, b_ref, w_ref, xwo_ref, yo_ref):
    dinv = _dinv_of(deg2_ref[...])
    agg = (s_ref[0] + s_ref[1]) * dinv[:, None]
    agg += xw_ref[...] * (dinv * dinv)[:, None]
    h = jax.nn.relu(agg + b_ref[...])
    xw = jnp.dot(h, w_ref[...], precision=_PREC)
    xwo_ref[...] = xw
    yo_ref[...] = xw * dinv[:, None]


def _tc_step(s, xw, deg2, b_prev, w_next):
    grid = (N // BT,)
    return pl.pallas_call(
        _step_body,
        grid=grid,
        in_specs=[
            pl.BlockSpec((NC, BT, F), lambda i: (0, i, 0)),
            pl.BlockSpec((BT, F), lambda i: (i, 0)),
            pl.BlockSpec((2, BT, 16), lambda i: (0, i, 0)),
            pl.BlockSpec((1, F), lambda i: (0, 0)),
            pl.BlockSpec((F, F), lambda i: (0, 0)),
        ],
        out_specs=[
            pl.BlockSpec((BT, F), lambda i: (i, 0)),
            pl.BlockSpec((BT, F), lambda i: (i, 0)),
        ],
        out_shape=[
            jax.ShapeDtypeStruct((N, F), jnp.float32),
            jax.ShapeDtypeStruct((N, F), jnp.float32),
        ],
    )(s, xw, deg2, b_prev, w_next)


GB_SK = 4


def _head_body(s1_ref, xw1_ref, d1_ref, s2_ref, xw2_ref, d2_ref, bg_ref,
               watt_ref, waff_ref, ws1_ref, bs1_ref, ws2_ref, bs2_ref,
               ged_ref, sim_ref):
    def gblk(s_ref, xw_ref, d_ref):
        dv = lax.rsqrt(d_ref[0, :, :, 0] + d_ref[1, :, :, 0] + 1.0)
        g = (s_ref[0] + s_ref[1]) * dv[:, :, None]
        g += xw_ref[...] * (dv * dv)[:, :, None]
        return g + bg_ref[...]

    g1 = gblk(s1_ref, xw1_ref, d1_ref)
    g2 = gblk(s2_ref, xw2_ref, d2_ref)

    def pool(hb):
        m = jnp.mean(hb, axis=0, keepdims=True)
        c = jnp.tanh(jnp.dot(m, watt_ref[...], precision=_PREC))
        a = jax.nn.sigmoid(
            lax.dot_general(hb, c, (((1,), (1,)), ((), ())),
                            precision=_PREC))
        return lax.dot_general(a, hb, (((0,), (0,)), ((), ())),
                               precision=_PREC)

    for i in range(GB_SK):
        h1 = g1[i]
        h2 = g2[i]
        e = jnp.concatenate([pool(h1), pool(h2)], axis=1)
        t = jax.nn.relu(jnp.dot(e, ws1_ref[...], precision=_PREC)
                        + bs1_ref[...])
        ged_ref[i] = jax.nn.sigmoid(
            jnp.dot(t, ws2_ref[...], precision=_PREC) + bs2_ref[...])

        a1 = jnp.dot(h1, waff_ref[...], precision=_PREC)
        m0 = lax.dot_general(a1, h2, (((1,), (1,)), ((), ())),
                             precision=_PREC)

        def it(_, ls):
            mx2 = jnp.max(ls, axis=1, keepdims=True)
            ls = ls - (mx2 + jnp.log(jnp.sum(jnp.exp(ls - mx2), axis=1,
                                             keepdims=True)))
            mx1 = jnp.max(ls, axis=0, keepdims=True)
            ls = ls - (mx1 + jnp.log(jnp.sum(jnp.exp(ls - mx1), axis=0,
                                             keepdims=True)))
            return ls

        ls = lax.fori_loop(0, SK_ITERS, it, m0 * (1.0 / TAU))
        sim_ref[i] = jnp.exp(ls)


def _tc_head(s1, xw1, d1, s2, xw2, d2, bg3, w_att, w_aff, ws1, bs1, ws2,
             bs2):
    grid = (B // GB_SK,)
    node_spec = [
        pl.BlockSpec((NC, GB_SK, NPG, F), lambda i: (0, i, 0, 0)),
        pl.BlockSpec((GB_SK, NPG, F), lambda i: (i, 0, 0)),
        pl.BlockSpec((NC, GB_SK, NPG, 16), lambda i: (0, i, 0, 0)),
    ]
    return pl.pallas_call(
        _head_body,
        grid=grid,
        in_specs=node_spec + node_spec + [
            pl.BlockSpec((1, F), lambda i: (0, 0)),
            pl.BlockSpec((F, F), lambda i: (0, 0)),
            pl.BlockSpec((F, F), lambda i: (0, 0)),
            pl.BlockSpec((2 * F, 16), lambda i: (0, 0)),
            pl.BlockSpec((1, 16), lambda i: (0, 0)),
            pl.BlockSpec((16, 1), lambda i: (0, 0)),
            pl.BlockSpec((1, 1), lambda i: (0, 0)),
        ],
        out_specs=[
            pl.BlockSpec((GB_SK, 1, 1), lambda i: (i, 0, 0)),
            pl.BlockSpec((GB_SK, NPG, NPG), lambda i: (i, 0, 0)),
        ],
        out_shape=[
            jax.ShapeDtypeStruct((B, 1, 1), jnp.float32),
            jax.ShapeDtypeStruct((B, NPG, NPG), jnp.float32),
        ],
    )(s1, xw1, d1, s2, xw2, d2, bg3, w_att, w_aff, ws1, bs1, ws2, bs2)


def kernel(x1, cent_pe1, rw_pe1, edge_index1, x2, cent_pe2, rw_pe2,
           edge_index2, degree_emb, W_init, b_init, Wg1, bg1, Wg2, bg2,
           Wg3, bg3, W_att, W_aff, Ws1, bs1, Ws2, bs2):
    z16 = jnp.zeros((N, 16), jnp.float32)
    z128 = jnp.zeros((N, F), jnp.float32)

    sd1 = jnp.stack([edge_index1[0].reshape(NW, NCH, CH),
                     edge_index1[1].reshape(NW, NCH, CH)], axis=2)
    sd2 = jnp.stack([edge_index2[0].reshape(NW, NCH, CH),
                     edge_index2[1].reshape(NW, NCH, CH)], axis=2)
    dsts_all = jnp.stack([edge_index1[1].reshape(NW, NCH, CH),
                          edge_index2[1].reshape(NW, NCH, CH)])

    deg_out = _sc_degrees(dsts_all, z16)
    deg_1 = deg_out[:, 0]
    deg_2 = deg_out[:, 1]

    b_init2 = b_init.reshape(1, F)
    bg1_2 = bg1.reshape(1, F)
    bg2_2 = bg2.reshape(1, F)
    bg3_2 = bg3.reshape(1, F)

    def conv_chain(x, cent, rw, deg2, sd):
        xw1, y1 = _tc_init(x, cent, rw, deg2, degree_emb, W_init, b_init2,
                           Wg1)
        s1 = _sc_scatter_rows(y1, sd, z128)
        xw2, y2 = _tc_step(s1, xw1, deg2, bg1_2, Wg2)
        s2 = _sc_scatter_rows(y2, sd, z128)
        xw3, y3 = _tc_step(s2, xw2, deg2, bg2_2, Wg3)
        s3 = _sc_scatter_rows(y3, sd, z128)
        return s3, xw3

    s3_1, xw3_1 = conv_chain(x1, cent_pe1, rw_pe1, deg_1, sd1)
    s3_2, xw3_2 = conv_chain(x2, cent_pe2, rw_pe2, deg_2, sd2)

    ged, sim = _tc_head(
        s3_1.reshape(NC, B, NPG, F), xw3_1.reshape(B, NPG, F),
        deg_1.reshape(NC, B, NPG, 16),
        s3_2.reshape(NC, B, NPG, F), xw3_2.reshape(B, NPG, F),
        deg_2.reshape(NC, B, NPG, 16),
        bg3_2, W_att, W_aff, Ws1, bs1.reshape(1, 16), Ws2,
        bs2.reshape(1, 1))
    return (ged.reshape(-1), sim, sim, sim)


# speedup vs baseline: 1.0146x; 1.0146x over previous
import functools

import jax
import jax.numpy as jnp
from jax import lax
from jax.experimental import pallas as pl
from jax.experimental.pallas import tpu as pltpu
from jax.experimental.pallas import tpu_sc as plsc

N = 10000
E = 320000
B = 100
NPG = 100
DX = 96
MAXDEG = 16
RW = 16
F = 128
TAU = 0.005
SK_ITERS = 10

NC = 2
NS = 16
NW = NC * NS
EPW = E // NW
CH = 100
NCH = EPW // CH
NT = NCH // 2
NTT = NCH // 3
RPS = 624
TAIL = N - NS * RPS

def _mesh():
    return plsc.VectorSubcoreMesh(core_axis_name="c", subcore_axis_name="s",
                                  num_cores=NC, num_subcores=NS)


def _deg_body(dsts_hbm, z_hbm, out_hbm, idx_v, ones_v, acc1, acc2,
              sem_a, sem_b):
    cid = lax.axis_index("c")
    sid = lax.axis_index("s")
    wid = sid * NC + cid

    @pl.when(sid == 0)
    def _zero():
        pltpu.sync_copy(z_hbm, acc1)
        pltpu.sync_copy(z_hbm, acc2)

    def _fill_ones(i, carry):
        ones_v[i] = jnp.ones((16,), jnp.float32)
        return carry

    lax.fori_loop(0, CH, _fill_ones, 0)
    plsc.subcore_barrier()

    for g, acc in ((0, acc1), (1, acc2)):
        pltpu.sync_copy(dsts_hbm.at[g, wid], idx_v)

        def _step(t, carry, acc=acc):
            @pl.when(t > 0)
            def _wa():
                pltpu.make_async_copy(
                    ones_v, acc.at[idx_v.at[2 * t - 2]], sem_a).wait()

            pltpu.async_copy(ones_v, acc.at[idx_v.at[2 * t]], sem_a,
                             add=True)

            @pl.when(t > 0)
            def _wb():
                pltpu.make_async_copy(
                    ones_v, acc.at[idx_v.at[2 * t - 1]], sem_b).wait()

            pltpu.async_copy(ones_v, acc.at[idx_v.at[2 * t + 1]], sem_b,
                             add=True)
            return carry

        lax.fori_loop(0, NT, _step, 0)
        pltpu.make_async_copy(ones_v, acc.at[idx_v.at[NCH - 2]],
                              sem_a).wait()
        pltpu.make_async_copy(ones_v, acc.at[idx_v.at[NCH - 1]],
                              sem_b).wait()

    plsc.subcore_barrier()
    base = pl.multiple_of(sid * RPS, 8)
    pltpu.sync_copy(acc1.at[pl.ds(base, RPS)],
                    out_hbm.at[cid, 0, pl.ds(base, RPS)])
    pltpu.sync_copy(acc2.at[pl.ds(base, RPS)],
                    out_hbm.at[cid, 1, pl.ds(base, RPS)])

    @pl.when(sid == 0)
    def _tail():
        pltpu.sync_copy(acc1.at[pl.ds(NS * RPS, TAIL)],
                        out_hbm.at[cid, 0, pl.ds(NS * RPS, TAIL)])
        pltpu.sync_copy(acc2.at[pl.ds(NS * RPS, TAIL)],
                        out_hbm.at[cid, 1, pl.ds(NS * RPS, TAIL)])


def _sc_degrees(dsts, z16):
    return pl.kernel(
        _deg_body,
        out_type=jax.ShapeDtypeStruct((NC, 2, N, 16), jnp.float32),
        mesh=_mesh(),
        scratch_types=[
            pltpu.VMEM((NCH, CH), jnp.int32),
            pltpu.VMEM((CH, 16), jnp.float32),
            pltpu.VMEM_SHARED((N, 16), jnp.float32),
            pltpu.VMEM_SHARED((N, 16), jnp.float32),
            pltpu.SemaphoreType.DMA,
            pltpu.SemaphoreType.DMA,
        ],
    )(dsts, z16)


def _scatter_body(y_hbm, srcs_hbm, dsts_hbm, z_hbm, out_hbm,
                  sidx, db0, db1, rows0, rows1, acc, gs0, gs1, ds0, ds1):
    cid = lax.axis_index("c")
    sid = lax.axis_index("s")
    wid = sid * NC + cid

    @pl.when(sid == 0)
    def _zero():
        pltpu.sync_copy(z_hbm, acc)

    pltpu.sync_copy(srcs_hbm.at[wid], sidx)
    plsc.subcore_barrier()

    pltpu.async_copy(y_hbm.at[sidx.at[0]], rows0, gs0)
    pltpu.async_copy(dsts_hbm.at[wid, pl.ds(0, 1)], db0, ds0)

    def _step(t, carry):
        j0 = 2 * t
        pltpu.async_copy(y_hbm.at[sidx.at[j0 + 1]], rows1, gs1)
        pltpu.async_copy(dsts_hbm.at[wid, pl.ds(j0 + 1, 1)], db1, ds1)
        pltpu.make_async_copy(y_hbm.at[sidx.at[j0]], rows0, gs0).wait()
        pltpu.make_async_copy(dsts_hbm.at[wid, pl.ds(j0, 1)], db0,
                              ds0).wait()
        pltpu.sync_copy(rows0, acc.at[db0.at[0]], add=True)

        @pl.when(t < NT - 1)
        def _pref():
            pltpu.async_copy(y_hbm.at[sidx.at[j0 + 2]], rows0, gs0)
            pltpu.async_copy(dsts_hbm.at[wid, pl.ds(j0 + 2, 1)], db0, ds0)

        pltpu.make_async_copy(y_hbm.at[sidx.at[j0 + 1]], rows1, gs1).wait()
        pltpu.make_async_copy(dsts_hbm.at[wid, pl.ds(j0 + 1, 1)], db1,
                              ds1).wait()
        pltpu.sync_copy(rows1, acc.at[db1.at[0]], add=True)
        return carry

    lax.fori_loop(0, NT, _step, 0)

    plsc.subcore_barrier()
    base = pl.multiple_of(sid * RPS, 8)
    pltpu.sync_copy(acc.at[pl.ds(base, RPS)],
                    out_hbm.at[cid, pl.ds(base, RPS)])

    @pl.when(sid == 0)
    def _tail():
        pltpu.sync_copy(acc.at[pl.ds(NS * RPS, TAIL)],
                        out_hbm.at[cid, pl.ds(NS * RPS, TAIL)])


def _sc_scatter_rows(y, srcs, dsts, z128):
    return pl.kernel(
        _scatter_body,
        out_type=jax.ShapeDtypeStruct((NC, N, F), jnp.float32),
        mesh=_mesh(),
        scratch_types=[
            pltpu.VMEM((NCH, CH), jnp.int32),
            pltpu.VMEM((1, CH), jnp.int32),
            pltpu.VMEM((1, CH), jnp.int32),
            pltpu.VMEM((CH, F), jnp.float32),
            pltpu.VMEM((CH, F), jnp.float32),
            pltpu.VMEM_SHARED((N, F), jnp.float32),
            pltpu.SemaphoreType.DMA,
            pltpu.SemaphoreType.DMA,
            pltpu.SemaphoreType.DMA,
            pltpu.SemaphoreType.DMA,
        ],
    )(y, srcs, dsts, z128)


BT = 2000
_PREC = lax.Precision.HIGHEST


def _dinv_of(deg2_blk):
    d = deg2_blk[0, :, 0] + deg2_blk[1, :, 0] + 1.0
    return lax.rsqrt(d)


def _init_body(x_ref, cent_ref, rw_ref, deg2_ref, demb_ref, wi_ref, bi_ref,
               wg_ref, xw_ref, y_ref):
    dinv = _dinv_of(deg2_ref[...])
    iot = lax.broadcasted_iota(jnp.int32, (1, MAXDEG), 1)
    oh = (cent_ref[...] == iot).astype(jnp.float32)
    h = jnp.dot(x_ref[...], wi_ref[0:DX, :], precision=_PREC)
    h += jnp.dot(jnp.dot(oh, demb_ref[...], precision=_PREC),
                 wi_ref[DX:DX + MAXDEG, :], precision=_PREC)
    h += jnp.dot(rw_ref[...], wi_ref[DX + MAXDEG:, :], precision=_PREC)
    h = jax.nn.relu(h + bi_ref[...])
    xw = jnp.dot(h, wg_ref[...], precision=_PREC)
    xw_ref[...] = xw
    y_ref[...] = xw * dinv[:, None]


def _tc_init(x, cent, rw, deg2, demb, w_init, b_init, wg1):
    grid = (N // BT,)
    return pl.pallas_call(
        _init_body,
        grid=grid,
        in_specs=[
            pl.BlockSpec((BT, DX), lambda i: (i, 0)),
            pl.BlockSpec((BT, 1), lambda i: (i, 0)),
            pl.BlockSpec((BT, RW), lambda i: (i, 0)),
            pl.BlockSpec((2, BT, 16), lambda i: (0, i, 0)),
            pl.BlockSpec((MAXDEG, MAXDEG), lambda i: (0, 0)),
            pl.BlockSpec((DX + MAXDEG + RW, F), lambda i: (0, 0)),
            pl.BlockSpec((1, F), lambda i: (0, 0)),
            pl.BlockSpec((F, F), lambda i: (0, 0)),
        ],
        out_specs=[
            pl.BlockSpec((BT, F), lambda i: (i, 0)),
            pl.BlockSpec((BT, F), lambda i: (i, 0)),
        ],
        out_shape=[
            jax.ShapeDtypeStruct((N, F), jnp.float32),
            jax.ShapeDtypeStruct((N, F), jnp.float32),
        ],
    )(x, cent, rw, deg2, demb, w_init, b_init, wg1)


def _step_body(s_ref, xw_ref, deg2_ref, b_ref, w_ref, xwo_ref, yo_ref):
    dinv = _dinv_of(deg2_ref[...])
    agg = (s_ref[0] + s_ref[1]) * dinv[:, None]
    agg += xw_ref[...] * (dinv * dinv)[:, None]
    h = jax.nn.relu(agg + b_ref[...])
    xw = jnp.dot(h, w_ref[...], precision=_PREC)
    xwo_ref[...] = xw
    yo_ref[...] = xw * dinv[:, None]


def _tc_step(s, xw, deg2, b_prev, w_next):
    grid = (N // BT,)
    return pl.pallas_call(
        _step_body,
        grid=grid,
        in_specs=[
            pl.BlockSpec((NC, BT, F), lambda i: (0, i, 0)),
            pl.BlockSpec((BT, F), lambda i: (i, 0)),
            pl.BlockSpec((2, BT, 16), lambda i: (0, i, 0)),
            pl.BlockSpec((1, F), lambda i: (0, 0)),
            pl.BlockSpec((F, F), lambda i: (0, 0)),
        ],
        out_specs=[
            pl.BlockSpec((BT, F), lambda i: (i, 0)),
            pl.BlockSpec((BT, F), lambda i: (i, 0)),
        ],
        out_shape=[
            jax.ShapeDtypeStruct((N, F), jnp.float32),
            jax.ShapeDtypeStruct((N, F), jnp.float32),
        ],
    )(s, xw, deg2, b_prev, w_next)


GB_SK = 4


def _head_body(s1_ref, xw1_ref, d1_ref, s2_ref, xw2_ref, d2_ref, bg_ref,
               watt_ref, waff_ref, ws1_ref, bs1_ref, ws2_ref, bs2_ref,
               ged_ref, sim_ref):
    def gblk(s_ref, xw_ref, d_ref):
        dv = lax.rsqrt(d_ref[0, :, :, 0] + d_ref[1, :, :, 0] + 1.0)
        g = (s_ref[0] + s_ref[1]) * dv[:, :, None]
        g += xw_ref[...] * (dv * dv)[:, :, None]
        return g + bg_ref[...]

    g1 = gblk(s1_ref, xw1_ref, d1_ref)
    g2 = gblk(s2_ref, xw2_ref, d2_ref)

    def pool(hb):
        m = jnp.mean(hb, axis=0, keepdims=True)
        c = jnp.tanh(jnp.dot(m, watt_ref[...], precision=_PREC))
        a = jax.nn.sigmoid(
            lax.dot_general(hb, c, (((1,), (1,)), ((), ())),
                            precision=_PREC))
        return lax.dot_general(a, hb, (((0,), (0,)), ((), ())),
                               precision=_PREC)

    for i in range(GB_SK):
        h1 = g1[i]
        h2 = g2[i]
        e = jnp.concatenate([pool(h1), pool(h2)], axis=1)
        t = jax.nn.relu(jnp.dot(e, ws1_ref[...], precision=_PREC)
                        + bs1_ref[...])
        ged_ref[i] = jax.nn.sigmoid(
            jnp.dot(t, ws2_ref[...], precision=_PREC) + bs2_ref[...])

        a1 = jnp.dot(h1, waff_ref[...], precision=_PREC)
        m0 = lax.dot_general(a1, h2, (((1,), (1,)), ((), ())),
                             precision=_PREC)

        def it(_, ls):
            mx2 = jnp.max(ls, axis=1, keepdims=True)
            ls = ls - (mx2 + jnp.log(jnp.sum(jnp.exp(ls - mx2), axis=1,
                                             keepdims=True)))
            mx1 = jnp.max(ls, axis=0, keepdims=True)
            ls = ls - (mx1 + jnp.log(jnp.sum(jnp.exp(ls - mx1), axis=0,
                                             keepdims=True)))
            return ls

        ls = lax.fori_loop(0, SK_ITERS, it, m0 * (1.0 / TAU))
        sim_ref[i] = jnp.exp(ls)


def _tc_head(s1, xw1, d1, s2, xw2, d2, bg3, w_att, w_aff, ws1, bs1, ws2,
             bs2):
    grid = (B // GB_SK,)
    node_spec = [
        pl.BlockSpec((NC, GB_SK, NPG, F), lambda i: (0, i, 0, 0)),
        pl.BlockSpec((GB_SK, NPG, F), lambda i: (i, 0, 0)),
        pl.BlockSpec((NC, GB_SK, NPG, 16), lambda i: (0, i, 0, 0)),
    ]
    return pl.pallas_call(
        _head_body,
        grid=grid,
        in_specs=node_spec + node_spec + [
            pl.BlockSpec((1, F), lambda i: (0, 0)),
            pl.BlockSpec((F, F), lambda i: (0, 0)),
            pl.BlockSpec((F, F), lambda i: (0, 0)),
            pl.BlockSpec((2 * F, 16), lambda i: (0, 0)),
            pl.BlockSpec((1, 16), lambda i: (0, 0)),
            pl.BlockSpec((16, 1), lambda i: (0, 0)),
            pl.BlockSpec((1, 1), lambda i: (0, 0)),
        ],
        out_specs=[
            pl.BlockSpec((GB_SK, 1, 1), lambda i: (i, 0, 0)),
            pl.BlockSpec((GB_SK, NPG, NPG), lambda i: (i, 0, 0)),
        ],
        out_shape=[
            jax.ShapeDtypeStruct((B, 1, 1), jnp.float32),
            jax.ShapeDtypeStruct((B, NPG, NPG), jnp.float32),
        ],
    )(s1, xw1, d1, s2, xw2, d2, bg3, w_att, w_aff, ws1, bs1, ws2, bs2)


def kernel(x1, cent_pe1, rw_pe1, edge_index1, x2, cent_pe2, rw_pe2,
           edge_index2, degree_emb, W_init, b_init, Wg1, bg1, Wg2, bg2,
           Wg3, bg3, W_att, W_aff, Ws1, bs1, Ws2, bs2):
    z16 = jnp.zeros((N, 16), jnp.float32)
    z128 = jnp.zeros((N, F), jnp.float32)

    srcs1 = edge_index1[0].reshape(NW, NCH, CH)
    dsts1 = edge_index1[1].reshape(NW, NCH, CH)
    srcs2 = edge_index2[0].reshape(NW, NCH, CH)
    dsts2 = edge_index2[1].reshape(NW, NCH, CH)
    dsts_all = jnp.stack([dsts1, dsts2])

    deg_out = _sc_degrees(dsts_all, z16)
    deg_1 = deg_out[:, 0]
    deg_2 = deg_out[:, 1]

    b_init2 = b_init.reshape(1, F)
    bg1_2 = bg1.reshape(1, F)
    bg2_2 = bg2.reshape(1, F)
    bg3_2 = bg3.reshape(1, F)

    def conv_chain(x, cent, rw, deg2, srcs, dsts):
        xw1, y1 = _tc_init(x, cent, rw, deg2, degree_emb, W_init, b_init2,
                           Wg1)
        s1 = _sc_scatter_rows(y1, srcs, dsts, z128)
        xw2, y2 = _tc_step(s1, xw1, deg2, bg1_2, Wg2)
        s2 = _sc_scatter_rows(y2, srcs, dsts, z128)
        xw3, y3 = _tc_step(s2, xw2, deg2, bg2_2, Wg3)
        s3 = _sc_scatter_rows(y3, srcs, dsts, z128)
        return s3, xw3

    s3_1, xw3_1 = conv_chain(x1, cent_pe1, rw_pe1, deg_1, srcs1, dsts1)
    s3_2, xw3_2 = conv_chain(x2, cent_pe2, rw_pe2, deg_2, srcs2, dsts2)

    ged, sim = _tc_head(
        s3_1.reshape(NC, B, NPG, F), xw3_1.reshape(B, NPG, F),
        deg_1.reshape(NC, B, NPG, 16),
        s3_2.reshape(NC, B, NPG, F), xw3_2.reshape(B, NPG, F),
        deg_2.reshape(NC, B, NPG, 16),
        bg3_2, W_att, W_aff, Ws1, bs1.reshape(1, 16), Ws2,
        bs2.reshape(1, 1))
    return (ged.reshape(-1), sim, sim, sim)


# speedup vs baseline: 1.2182x; 1.2007x over previous
import functools

import jax
import jax.numpy as jnp
from jax import lax
from jax.experimental import pallas as pl
from jax.experimental.pallas import tpu as pltpu
from jax.experimental.pallas import tpu_sc as plsc

N = 10000
E = 320000
B = 100
NPG = 100
DX = 96
MAXDEG = 16
RW = 16
F = 128
TAU = 0.005
SK_ITERS = 10

NC = 2
NS = 16
NW = NC * NS
EPW = E // NW
CH = 100
NCH = EPW // CH
NT = NCH // 2
NTT = NCH // 3
RPS = 624
TAIL = N - NS * RPS

def _mesh():
    return plsc.VectorSubcoreMesh(core_axis_name="c", subcore_axis_name="s",
                                  num_cores=NC, num_subcores=NS)


def _deg_body(dsts_hbm, z_hbm, out_hbm, idx_v, ones_v, acc1, acc2,
              sem_a, sem_b):
    cid = lax.axis_index("c")
    sid = lax.axis_index("s")
    wid = sid * NC + cid

    @pl.when(sid == 0)
    def _zero():
        pltpu.sync_copy(z_hbm, acc1)
        pltpu.sync_copy(z_hbm, acc2)

    def _fill_ones(i, carry):
        ones_v[i] = jnp.ones((16,), jnp.float32)
        return carry

    lax.fori_loop(0, CH, _fill_ones, 0)
    plsc.subcore_barrier()

    for g, acc in ((0, acc1), (1, acc2)):
        pltpu.sync_copy(dsts_hbm.at[g, wid], idx_v)

        def _step(t, carry, acc=acc):
            @pl.when(t > 0)
            def _wa():
                pltpu.make_async_copy(
                    ones_v, acc.at[idx_v.at[2 * t - 2]], sem_a).wait()

            pltpu.async_copy(ones_v, acc.at[idx_v.at[2 * t]], sem_a,
                             add=True)

            @pl.when(t > 0)
            def _wb():
                pltpu.make_async_copy(
                    ones_v, acc.at[idx_v.at[2 * t - 1]], sem_b).wait()

            pltpu.async_copy(ones_v, acc.at[idx_v.at[2 * t + 1]], sem_b,
                             add=True)
            return carry

        lax.fori_loop(0, NT, _step, 0)
        pltpu.make_async_copy(ones_v, acc.at[idx_v.at[NCH - 2]],
                              sem_a).wait()
        pltpu.make_async_copy(ones_v, acc.at[idx_v.at[NCH - 1]],
                              sem_b).wait()

    plsc.subcore_barrier()
    base = pl.multiple_of(sid * RPS, 8)
    pltpu.sync_copy(acc1.at[pl.ds(base, RPS)],
                    out_hbm.at[cid, 0, pl.ds(base, RPS)])
    pltpu.sync_copy(acc2.at[pl.ds(base, RPS)],
                    out_hbm.at[cid, 1, pl.ds(base, RPS)])

    @pl.when(sid == 0)
    def _tail():
        pltpu.sync_copy(acc1.at[pl.ds(NS * RPS, TAIL)],
                        out_hbm.at[cid, 0, pl.ds(NS * RPS, TAIL)])
        pltpu.sync_copy(acc2.at[pl.ds(NS * RPS, TAIL)],
                        out_hbm.at[cid, 1, pl.ds(NS * RPS, TAIL)])


def _sc_degrees(dsts, z16):
    return pl.kernel(
        _deg_body,
        out_type=jax.ShapeDtypeStruct((NC, 2, N, 16), jnp.float32),
        mesh=_mesh(),
        scratch_types=[
            pltpu.VMEM((NCH, CH), jnp.int32),
            pltpu.VMEM((CH, 16), jnp.float32),
            pltpu.VMEM_SHARED((N, 16), jnp.float32),
            pltpu.VMEM_SHARED((N, 16), jnp.float32),
            pltpu.SemaphoreType.DMA,
            pltpu.SemaphoreType.DMA,
        ],
    )(dsts, z16)


def _scatter_body(y_hbm, srcs_hbm, dsts_hbm, z_hbm, out_hbm,
                  sidx, db0, db1, rows0, rows1, acc, gs0, gs1, ds0, ds1):
    cid = lax.axis_index("c")
    sid = lax.axis_index("s")
    wid = sid * NC + cid

    @pl.when(sid == 0)
    def _zero():
        pltpu.sync_copy(z_hbm, acc)

    pltpu.sync_copy(srcs_hbm.at[wid], sidx)
    plsc.subcore_barrier()

    pltpu.async_copy(y_hbm.at[sidx.at[0]], rows0, gs0)
    pltpu.async_copy(dsts_hbm.at[wid, pl.ds(0, 1)], db0, ds0)

    def _step(t, carry):
        j0 = 2 * t
        pltpu.async_copy(y_hbm.at[sidx.at[j0 + 1]], rows1, gs1)
        pltpu.async_copy(dsts_hbm.at[wid, pl.ds(j0 + 1, 1)], db1, ds1)
        pltpu.make_async_copy(y_hbm.at[sidx.at[j0]], rows0, gs0).wait()
        pltpu.make_async_copy(dsts_hbm.at[wid, pl.ds(j0, 1)], db0,
                              ds0).wait()
        pltpu.sync_copy(rows0, acc.at[db0.at[0]], add=True)

        @pl.when(t < NT - 1)
        def _pref():
            pltpu.async_copy(y_hbm.at[sidx.at[j0 + 2]], rows0, gs0)
            pltpu.async_copy(dsts_hbm.at[wid, pl.ds(j0 + 2, 1)], db0, ds0)

        pltpu.make_async_copy(y_hbm.at[sidx.at[j0 + 1]], rows1, gs1).wait()
        pltpu.make_async_copy(dsts_hbm.at[wid, pl.ds(j0 + 1, 1)], db1,
                              ds1).wait()
        pltpu.sync_copy(rows1, acc.at[db1.at[0]], add=True)
        return carry

    lax.fori_loop(0, NT, _step, 0)

    plsc.subcore_barrier()
    base = pl.multiple_of(sid * RPS, 8)
    pltpu.sync_copy(acc.at[pl.ds(base, RPS)],
                    out_hbm.at[cid, pl.ds(base, RPS)])

    @pl.when(sid == 0)
    def _tail():
        pltpu.sync_copy(acc.at[pl.ds(NS * RPS, TAIL)],
                        out_hbm.at[cid, pl.ds(NS * RPS, TAIL)])


def _sc_scatter_rows(y, srcs, dsts, z128):
    return pl.kernel(
        _scatter_body,
        out_type=jax.ShapeDtypeStruct((NC, N, F), jnp.float32),
        mesh=_mesh(),
        scratch_types=[
            pltpu.VMEM((NCH, CH), jnp.int32),
            pltpu.VMEM((1, CH), jnp.int32),
            pltpu.VMEM((1, CH), jnp.int32),
            pltpu.VMEM((CH, F), jnp.float32),
            pltpu.VMEM((CH, F), jnp.float32),
            pltpu.VMEM_SHARED((N, F), jnp.float32),
            pltpu.SemaphoreType.DMA,
            pltpu.SemaphoreType.DMA,
            pltpu.SemaphoreType.DMA,
            pltpu.SemaphoreType.DMA,
        ],
    )(y, srcs, dsts, z128)


BT = 2000
_PREC = lax.Precision.HIGHEST


def _dinv_of(deg2_blk):
    d = deg2_blk[0, :, 0] + deg2_blk[1, :, 0] + 1.0
    return lax.rsqrt(d)


def _init_body(x_ref, cent_ref, rw_ref, deg2_ref, demb_ref, wi_ref, bi_ref,
               wg_ref, xw_ref, y_ref):
    dinv = _dinv_of(deg2_ref[...])
    iot = lax.broadcasted_iota(jnp.int32, (1, MAXDEG), 1)
    oh = (cent_ref[...] == iot).astype(jnp.float32)
    h = jnp.dot(x_ref[...], wi_ref[0:DX, :], precision=_PREC)
    h += jnp.dot(jnp.dot(oh, demb_ref[...], precision=_PREC),
                 wi_ref[DX:DX + MAXDEG, :], precision=_PREC)
    h += jnp.dot(rw_ref[...], wi_ref[DX + MAXDEG:, :], precision=_PREC)
    h = jax.nn.relu(h + bi_ref[...])
    xw = jnp.dot(h, wg_ref[...], precision=_PREC)
    xw_ref[...] = xw
    y_ref[...] = xw * dinv[:, None]


def _tc_init(x, cent, rw, deg2, demb, w_init, b_init, wg1):
    grid = (N // BT,)
    return pl.pallas_call(
        _init_body,
        grid=grid,
        in_specs=[
            pl.BlockSpec((BT, DX), lambda i: (i, 0)),
            pl.BlockSpec((BT, 1), lambda i: (i, 0)),
            pl.BlockSpec((BT, RW), lambda i: (i, 0)),
            pl.BlockSpec((2, BT, 16), lambda i: (0, i, 0)),
            pl.BlockSpec((MAXDEG, MAXDEG), lambda i: (0, 0)),
            pl.BlockSpec((DX + MAXDEG + RW, F), lambda i: (0, 0)),
            pl.BlockSpec((1, F), lambda i: (0, 0)),
            pl.BlockSpec((F, F), lambda i: (0, 0)),
        ],
        out_specs=[
            pl.BlockSpec((BT, F), lambda i: (i, 0)),
            pl.BlockSpec((BT, F), lambda i: (i, 0)),
        ],
        out_shape=[
            jax.ShapeDtypeStruct((N, F), jnp.float32),
            jax.ShapeDtypeStruct((N, F), jnp.float32),
        ],
    )(x, cent, rw, deg2, demb, w_init, b_init, wg1)


def _step_body(s_ref, xw_ref, deg2_ref, b_ref, w_ref, xwo_ref, yo_ref):
    dinv = _dinv_of(deg2_ref[...])
    agg = (s_ref[0] + s_ref[1]) * dinv[:, None]
    agg += xw_ref[...] * (dinv * dinv)[:, None]
    h = jax.nn.relu(agg + b_ref[...])
    xw = jnp.dot(h, w_ref[...], precision=_PREC)
    xwo_ref[...] = xw
    yo_ref[...] = xw * dinv[:, None]


def _tc_step(s, xw, deg2, b_prev, w_next):
    grid = (N // BT,)
    return pl.pallas_call(
        _step_body,
        grid=grid,
        in_specs=[
            pl.BlockSpec((NC, BT, F), lambda i: (0, i, 0)),
            pl.BlockSpec((BT, F), lambda i: (i, 0)),
            pl.BlockSpec((2, BT, 16), lambda i: (0, i, 0)),
            pl.BlockSpec((1, F), lambda i: (0, 0)),
            pl.BlockSpec((F, F), lambda i: (0, 0)),
        ],
        out_specs=[
            pl.BlockSpec((BT, F), lambda i: (i, 0)),
            pl.BlockSpec((BT, F), lambda i: (i, 0)),
        ],
        out_shape=[
            jax.ShapeDtypeStruct((N, F), jnp.float32),
            jax.ShapeDtypeStruct((N, F), jnp.float32),
        ],
    )(s, xw, deg2, b_prev, w_next)


def _final_body(s_ref, xw_ref, deg2_ref, b_ref, g_ref):
    dinv = _dinv_of(deg2_ref[...])
    agg = (s_ref[0] + s_ref[1]) * dinv[:, None]
    agg += xw_ref[...] * (dinv * dinv)[:, None]
    g_ref[...] = agg + b_ref[...]


def _tc_final(s, xw, deg2, b_prev):
    grid = (N // BT,)
    return pl.pallas_call(
        _final_body,
        grid=grid,
        in_specs=[
            pl.BlockSpec((NC, BT, F), lambda i: (0, i, 0)),
            pl.BlockSpec((BT, F), lambda i: (i, 0)),
            pl.BlockSpec((2, BT, 16), lambda i: (0, i, 0)),
            pl.BlockSpec((1, F), lambda i: (0, 0)),
        ],
        out_specs=pl.BlockSpec((BT, F), lambda i: (i, 0)),
        out_shape=jax.ShapeDtypeStruct((N, F), jnp.float32),
    )(s, xw, deg2, b_prev)


def _attn_body(g1_ref, g2_ref, watt_ref, ws1_ref, bs1_ref, ws2_ref, bs2_ref,
               ged_ref):
    def pool(hb):
        m = jnp.mean(hb, axis=1)
        c = jnp.tanh(jnp.dot(m, watt_ref[...], precision=_PREC))
        a = jax.nn.sigmoid(jnp.sum(hb * c[:, None, :], axis=2))
        return jnp.sum(hb * a[:, :, None], axis=1)

    e1 = pool(g1_ref[...])
    e2 = pool(g2_ref[...])
    s = jnp.concatenate([e1, e2], axis=1)
    t = jax.nn.relu(jnp.dot(s, ws1_ref[...], precision=_PREC) + bs1_ref[...])
    ged_ref[...] = jax.nn.sigmoid(
        jnp.dot(t, ws2_ref[...], precision=_PREC) + bs2_ref[...])


def _tc_attn_ged(g1r, g2r, w_att, ws1, bs1, ws2, bs2):
    return pl.pallas_call(
        _attn_body,
        grid=(1,),
        in_specs=[
            pl.BlockSpec((B, NPG, F), lambda i: (0, 0, 0)),
            pl.BlockSpec((B, NPG, F), lambda i: (0, 0, 0)),
            pl.BlockSpec((F, F), lambda i: (0, 0)),
            pl.BlockSpec((2 * F, 16), lambda i: (0, 0)),
            pl.BlockSpec((1, 16), lambda i: (0, 0)),
            pl.BlockSpec((16, 1), lambda i: (0, 0)),
            pl.BlockSpec((1, 1), lambda i: (0, 0)),
        ],
        out_specs=pl.BlockSpec((B, 1), lambda i: (0, 0)),
        out_shape=jax.ShapeDtypeStruct((B, 1), jnp.float32),
    )(g1r, g2r, w_att, ws1, bs1, ws2, bs2)


GB_SK = 4
BSK = GB_SK * NPG


def _sinkhorn_body(g1_ref, g2_ref, waff_ref, sim_ref):
    a1 = jnp.dot(g1_ref[...], waff_ref[...], precision=_PREC)
    h2 = g2_ref[...]
    for k in range(GB_SK):
        a1g = a1[k * NPG:(k + 1) * NPG]
        h2g = h2[k * NPG:(k + 1) * NPG]
        m0 = lax.dot_general(a1g, h2g, (((1,), (1,)), ((), ())),
                             precision=_PREC)
        ls = m0 * (1.0 / TAU)
        mx = jnp.max(ls, axis=1, keepdims=True)
        ls = ls - (mx + jnp.log(jnp.sum(jnp.exp(ls - mx), axis=1,
                                        keepdims=True)))
        ls = ls - jnp.log(jnp.sum(jnp.exp(ls), axis=0, keepdims=True))

        def it(_, ls):
            ls = ls - jnp.log(jnp.sum(jnp.exp(ls), axis=1, keepdims=True))
            ls = ls - jnp.log(jnp.sum(jnp.exp(ls), axis=0, keepdims=True))
            return ls

        ls = lax.fori_loop(0, SK_ITERS - 1, it, ls)
        sim_ref[k] = jnp.exp(ls)


def _tc_sinkhorn(g1, g2, w_aff):
    grid = (N // BSK,)
    return pl.pallas_call(
        _sinkhorn_body,
        grid=grid,
        in_specs=[
            pl.BlockSpec((BSK, F), lambda i: (i, 0)),
            pl.BlockSpec((BSK, F), lambda i: (i, 0)),
            pl.BlockSpec((F, F), lambda i: (0, 0)),
        ],
        out_specs=pl.BlockSpec((GB_SK, NPG, NPG), lambda i: (i, 0, 0)),
        out_shape=jax.ShapeDtypeStruct((B, NPG, NPG), jnp.float32),
    )(g1, g2, w_aff)


def kernel(x1, cent_pe1, rw_pe1, edge_index1, x2, cent_pe2, rw_pe2,
           edge_index2, degree_emb, W_init, b_init, Wg1, bg1, Wg2, bg2,
           Wg3, bg3, W_att, W_aff, Ws1, bs1, Ws2, bs2):
    z16 = jnp.zeros((N, 16), jnp.float32)
    z128 = jnp.zeros((N, F), jnp.float32)

    srcs1 = edge_index1[0].reshape(NW, NCH, CH)
    dsts1 = edge_index1[1].reshape(NW, NCH, CH)
    srcs2 = edge_index2[0].reshape(NW, NCH, CH)
    dsts2 = edge_index2[1].reshape(NW, NCH, CH)
    dsts_all = jnp.stack([dsts1, dsts2])

    deg_out = _sc_degrees(dsts_all, z16)
    deg_1 = deg_out[:, 0]
    deg_2 = deg_out[:, 1]

    b_init2 = b_init.reshape(1, F)
    bg1_2 = bg1.reshape(1, F)
    bg2_2 = bg2.reshape(1, F)
    bg3_2 = bg3.reshape(1, F)

    def conv_chain(x, cent, rw, deg2, srcs, dsts):
        xw1, y1 = _tc_init(x, cent, rw, deg2, degree_emb, W_init, b_init2,
                           Wg1)
        s1 = _sc_scatter_rows(y1, srcs, dsts, z128)
        xw2, y2 = _tc_step(s1, xw1, deg2, bg1_2, Wg2)
        s2 = _sc_scatter_rows(y2, srcs, dsts, z128)
        xw3, y3 = _tc_step(s2, xw2, deg2, bg2_2, Wg3)
        s3 = _sc_scatter_rows(y3, srcs, dsts, z128)
        return _tc_final(s3, xw3, deg2, bg3_2)

    g1 = conv_chain(x1, cent_pe1, rw_pe1, deg_1, srcs1, dsts1)
    g2 = conv_chain(x2, cent_pe2, rw_pe2, deg_2, srcs2, dsts2)

    ged = _tc_attn_ged(g1.reshape(B, NPG, F), g2.reshape(B, NPG, F),
                       W_att, Ws1, bs1.reshape(1, 16), Ws2,
                       bs2.reshape(1, 1))
    sim = _tc_sinkhorn(g1, g2, W_aff)
    return (ged.reshape(-1), sim, sim, sim)
